# baseline (speedup 1.0000x reference)
import functools

import jax
import jax.numpy as jnp
from jax.experimental import pallas as pl
from jax.experimental.pallas import tpu as pltpu
from jax.experimental.pallas import tpu_sc as plsc

B = 16384
F = 26
V = 1000
D = 64
H = 128

NC = 2
NS = 16
NW = NC * NS

ROWS_PER_WIN = 4
GW = ROWS_PER_WIN * F
BLK = 1024


def _project_tables(tables, W1):
    def body(t_ref, w_ref, o_ref):
        tb = t_ref[0].astype(jnp.bfloat16)
        o_ref[0] = jnp.dot(tb, w_ref[0], preferred_element_type=jnp.float32)

    proj = pl.pallas_call(
        body,
        grid=(F,),
        in_specs=[
            pl.BlockSpec((1, V, D), lambda f: (f, 0, 0)),
            pl.BlockSpec((1, D, H), lambda f: (f, 0, 0)),
        ],
        out_specs=pl.BlockSpec((1, V, H), lambda f: (f, 0, 0)),
        out_shape=jax.ShapeDtypeStruct((F, V, H), jnp.float32),
    )(tables, W1)
    return proj.reshape(F * V, H)


def _gather_sum(proj, flat_idx):
    n = flat_idx.shape[0]
    per_w = n // NW
    rows_w = per_w // F
    nwin = per_w // GW
    mesh = plsc.VectorSubcoreMesh(core_axis_name="core", subcore_axis_name="subcore")

    @functools.partial(
        pl.kernel,
        out_type=jax.ShapeDtypeStruct((B, H), jnp.float32),
        mesh=mesh,
        scratch_types=[
            pltpu.VMEM((per_w,), jnp.int32),
            pltpu.VMEM((GW, H), jnp.float32),
            pltpu.VMEM((GW, H), jnp.float32),
            pltpu.VMEM((rows_w, H), jnp.float32),
            pltpu.SemaphoreType.DMA,
            pltpu.SemaphoreType.DMA,
        ],
    )
    def gather_kernel(proj_hbm, idx_hbm, out_hbm, idx_v, buf0, buf1, obuf, sem0, sem1):
        bufs = (buf0, buf1)
        sems = (sem0, sem1)
        wid = jax.lax.axis_index("subcore") * NC + jax.lax.axis_index("core")
        base = wid * per_w
        pltpu.sync_copy(idx_hbm.at[pl.ds(base, per_w)], idx_v)

        def gather_start(win, buf, sem):
            pltpu.async_copy(proj_hbm.at[idx_v.at[pl.ds(win * GW, GW)]], buf, sem)

        def drain_sum(win, buf, sem):
            pltpu.make_async_copy(
                proj_hbm.at[idx_v.at[pl.ds(win * GW, GW)]], buf, sem
            ).wait()
            for r in range(ROWS_PER_WIN):
                for c in range(H // 16):
                    cs = pl.ds(c * 16, 16)
                    acc = buf[pl.ds(r * F, 1), cs]
                    for f in range(1, F):
                        acc = acc + buf[pl.ds(r * F + f, 1), cs]
                    obuf[pl.ds(win * ROWS_PER_WIN + r, 1), cs] = acc

        gather_start(0, buf0, sem0)

        @pl.loop(0, nwin, step=2)
        def _(wn):
            @pl.when(wn + 1 < nwin)
            def _():
                gather_start(wn + 1, bufs[1], sems[1])

            drain_sum(wn, bufs[0], sems[0])

            @pl.when(wn + 2 < nwin)
            def _():
                gather_start(wn + 2, bufs[0], sems[0])

            @pl.when(wn + 1 < nwin)
            def _():
                drain_sum(wn + 1, bufs[1], sems[1])

        pltpu.sync_copy(obuf, out_hbm.at[pl.ds(wid * rows_w, rows_w)])

    return gather_kernel(proj, flat_idx)


def _finish(h, b1, w2, b2):
    def body(h_ref, b1_ref, w2_ref, b2_ref, o_ref):
        a = jnp.maximum(h_ref[...] + b1_ref[...], 0.0).astype(jnp.bfloat16)
        o = jnp.dot(a, w2_ref[...], preferred_element_type=jnp.float32)
        o_ref[...] = o + b2_ref[...]

    return pl.pallas_call(
        body,
        grid=(B // BLK,),
        in_specs=[
            pl.BlockSpec((BLK, H), lambda i: (i, 0)),
            pl.BlockSpec((1, H), lambda i: (0, 0)),
            pl.BlockSpec((H, D), lambda i: (0, 0)),
            pl.BlockSpec((1, D), lambda i: (0, 0)),
        ],
        out_specs=pl.BlockSpec((BLK, D), lambda i: (i, 0)),
        out_shape=jax.ShapeDtypeStruct((B, D), jnp.float32),
    )(h, b1.reshape(1, H), w2, b2.reshape(1, D))


def kernel(idx, tables, W1, b1, W2, b2):
    idx = idx.astype(jnp.int32)
    flat_idx = (idx + (jnp.arange(F, dtype=jnp.int32) * V)[None, :]).reshape(B * F)
    w1 = W1.astype(jnp.bfloat16).reshape(F, D, H)
    proj = _project_tables(tables, w1)
    h = _gather_sum(proj, flat_idx)
    return _finish(h, b1, W2.astype(jnp.bfloat16), b2)


# speedup vs baseline: 1.8850x; 1.8850x over previous
import functools

import jax
import jax.numpy as jnp
from jax.experimental import pallas as pl
from jax.experimental.pallas import tpu as pltpu
from jax.experimental.pallas import tpu_sc as plsc

B = 16384
F = 26
V = 1000
D = 64
H = 128

NC = 2
NS = 16
NW = NC * NS

ROWS_PER_WIN = 4
GW = ROWS_PER_WIN * F
BLK = 1024


def _project_tables(tables, W1):
    def body(t_ref, w_ref, o_ref):
        tb = t_ref[0].astype(jnp.bfloat16)
        o_ref[0] = jnp.dot(tb, w_ref[0], preferred_element_type=jnp.float32)

    def body2(t_ref, w_ref, o_ref):
        tb = t_ref[0].astype(jnp.bfloat16)
        o_ref[...] = jnp.dot(tb, w_ref[0], preferred_element_type=jnp.float32)

    return pl.pallas_call(
        body2,
        grid=(F,),
        in_specs=[
            pl.BlockSpec((1, V, D), lambda f: (f, 0, 0)),
            pl.BlockSpec((1, D, H), lambda f: (f, 0, 0)),
        ],
        out_specs=pl.BlockSpec((V, H), lambda f: (f, 0)),
        out_shape=jax.ShapeDtypeStruct((F * V, H), jnp.float32),
    )(tables, W1)


def _gather_sum(proj, flat_idx, pat, zero):
    n = flat_idx.shape[0]
    per_w = n // NW
    rows_w = per_w // F
    nwin = per_w // GW
    mesh = plsc.VectorSubcoreMesh(core_axis_name="core", subcore_axis_name="subcore")

    @functools.partial(
        pl.kernel,
        out_type=jax.ShapeDtypeStruct((B, H), jnp.float32),
        mesh=mesh,
        scratch_types=[
            pltpu.VMEM((per_w,), jnp.int32),
            pltpu.VMEM((nwin, GW), jnp.int32),
            pltpu.VMEM((GW, H), jnp.float32),
            pltpu.VMEM((GW, H), jnp.float32),
            pltpu.VMEM_SHARED((NS * rows_w, H), jnp.float32),
            pltpu.SemaphoreType.DMA,
            pltpu.SemaphoreType.DMA,
        ],
    )
    def gather_kernel(proj_hbm, idx_hbm, pat_hbm, z_hbm, out_hbm,
                      idx_v, pat_v, buf0, buf1, acc, sem0, sem1):
        bufs = (buf0, buf1)
        sems = (sem0, sem1)
        sid = jax.lax.axis_index("subcore")
        wid = sid * NC + jax.lax.axis_index("core")
        base = wid * per_w
        pltpu.sync_copy(idx_hbm.at[pl.ds(base, per_w)], idx_v)
        pltpu.sync_copy(pat_hbm.at[sid], pat_v)
        pltpu.sync_copy(z_hbm, acc.at[pl.ds(sid * rows_w, rows_w)])

        def gather_start(win, buf, sem):
            pltpu.async_copy(proj_hbm.at[idx_v.at[pl.ds(win * GW, GW)]], buf, sem)

        def drain_add(win, buf, sem):
            pltpu.make_async_copy(
                proj_hbm.at[idx_v.at[pl.ds(win * GW, GW)]], buf, sem
            ).wait()
            pltpu.sync_copy(buf, acc.at[pat_v.at[win]], add=True)

        gather_start(0, buf0, sem0)

        @pl.loop(0, nwin, step=2)
        def _(wn):
            @pl.when(wn + 1 < nwin)
            def _():
                gather_start(wn + 1, bufs[1], sems[1])

            drain_add(wn, bufs[0], sems[0])

            @pl.when(wn + 2 < nwin)
            def _():
                gather_start(wn + 2, bufs[0], sems[0])

            @pl.when(wn + 1 < nwin)
            def _():
                drain_add(wn + 1, bufs[1], sems[1])

        pltpu.sync_copy(
            acc.at[pl.ds(sid * rows_w, rows_w)],
            out_hbm.at[pl.ds(wid * rows_w, rows_w)],
        )

    return gather_kernel(proj, flat_idx, pat, zero)


def _finish(h, b1, w2, b2):
    def body(h_ref, b1_ref, w2_ref, b2_ref, o_ref):
        a = jnp.maximum(h_ref[...] + b1_ref[...], 0.0).astype(jnp.bfloat16)
        o = jnp.dot(a, w2_ref[...], preferred_element_type=jnp.float32)
        o_ref[...] = o + b2_ref[...]

    return pl.pallas_call(
        body,
        grid=(B // BLK,),
        in_specs=[
            pl.BlockSpec((BLK, H), lambda i: (i, 0)),
            pl.BlockSpec((1, H), lambda i: (0, 0)),
            pl.BlockSpec((H, D), lambda i: (0, 0)),
            pl.BlockSpec((1, D), lambda i: (0, 0)),
        ],
        out_specs=pl.BlockSpec((BLK, D), lambda i: (i, 0)),
        out_shape=jax.ShapeDtypeStruct((B, D), jnp.float32),
    )(h, b1.reshape(1, H), w2, b2.reshape(1, D))


def kernel(idx, tables, W1, b1, W2, b2):
    idx = idx.astype(jnp.int32)
    flat_idx = (idx + (jnp.arange(F, dtype=jnp.int32) * V)[None, :]).reshape(B * F)
    w1 = W1.astype(jnp.bfloat16).reshape(F, D, H)
    proj = _project_tables(tables, w1)
    rows_w = B // NW
    nwin = rows_w // ROWS_PER_WIN
    pat = (
        (jnp.arange(NS, dtype=jnp.int32) * rows_w)[:, None, None]
        + (jnp.arange(nwin, dtype=jnp.int32) * ROWS_PER_WIN)[None, :, None]
        + (jnp.arange(GW, dtype=jnp.int32) // F)[None, None, :]
    )
    zero = jnp.zeros((rows_w, H), jnp.float32)
    h = _gather_sum(proj, flat_idx, pat, zero)
    return _finish(h, b1, W2.astype(jnp.bfloat16), b2)


# speedup vs baseline: 1.8881x; 1.0016x over previous
import functools

import jax
import jax.numpy as jnp
from jax.experimental import pallas as pl
from jax.experimental.pallas import tpu as pltpu
from jax.experimental.pallas import tpu_sc as plsc

B = 16384
F = 26
V = 1000
D = 64
H = 128

NC = 2
NS = 16
NW = NC * NS

ROWS_PER_WIN = 4
GW = ROWS_PER_WIN * F
BLK = 1024


def _project_tables(tables, W1):
    def body(t_ref, w_ref, o_ref):
        tb = t_ref[0].astype(jnp.bfloat16)
        o_ref[...] = jnp.dot(tb, w_ref[0], preferred_element_type=jnp.float32)

    return pl.pallas_call(
        body,
        grid=(F,),
        in_specs=[
            pl.BlockSpec((1, V, D), lambda f: (f, 0, 0)),
            pl.BlockSpec((1, D, H), lambda f: (f, 0, 0)),
        ],
        out_specs=pl.BlockSpec((V, H), lambda f: (f, 0)),
        out_shape=jax.ShapeDtypeStruct((F * V, H), jnp.float32),
    )(tables, W1)


def _gather_sum(proj, flat_idx, pat, zero):
    n = flat_idx.shape[0]
    per_w = n // NW
    rows_w = per_w // F
    nwin = per_w // GW
    mesh = plsc.VectorSubcoreMesh(core_axis_name="core", subcore_axis_name="subcore")

    @functools.partial(
        pl.kernel,
        out_type=jax.ShapeDtypeStruct((B, H), jnp.float32),
        mesh=mesh,
        scratch_types=[
            pltpu.VMEM((per_w,), jnp.int32),
            pltpu.VMEM((nwin, GW), jnp.int32),
            pltpu.VMEM((GW, H), jnp.float32),
            pltpu.VMEM((GW, H), jnp.float32),
            pltpu.VMEM_SHARED((NS * rows_w, H), jnp.float32),
            pltpu.SemaphoreType.DMA,
            pltpu.SemaphoreType.DMA,
        ],
    )
    def gather_kernel(proj_hbm, idx_hbm, pat_hbm, z_hbm, out_hbm,
                      idx_v, pat_v, buf0, buf1, acc, sem0, sem1):
        bufs = (buf0, buf1)
        sems = (sem0, sem1)
        sid = jax.lax.axis_index("subcore")
        wid = sid * NC + jax.lax.axis_index("core")
        base = wid * per_w
        pltpu.sync_copy(idx_hbm.at[pl.ds(base, per_w)], idx_v)
        pltpu.sync_copy(pat_hbm.at[sid], pat_v)
        pltpu.sync_copy(z_hbm, acc.at[pl.ds(sid * rows_w, rows_w)])

        def gather_start(win, buf, sem):
            pltpu.async_copy(proj_hbm.at[idx_v.at[pl.ds(win * GW, GW)]], buf, sem)

        def drain_add(win, buf, sem):
            pltpu.make_async_copy(
                proj_hbm.at[idx_v.at[pl.ds(win * GW, GW)]], buf, sem
            ).wait()
            pltpu.sync_copy(buf, acc.at[pat_v.at[win]], add=True)

        gather_start(0, buf0, sem0)

        @pl.loop(0, nwin, step=2)
        def _(wn):
            @pl.when(wn + 1 < nwin)
            def _():
                gather_start(wn + 1, bufs[1], sems[1])

            drain_add(wn, bufs[0], sems[0])

            @pl.when(wn + 2 < nwin)
            def _():
                gather_start(wn + 2, bufs[0], sems[0])

            @pl.when(wn + 1 < nwin)
            def _():
                drain_add(wn + 1, bufs[1], sems[1])

        pltpu.sync_copy(
            acc.at[pl.ds(sid * rows_w, rows_w)],
            out_hbm.at[pl.ds(wid * rows_w, rows_w)],
        )

    return gather_kernel(proj, flat_idx, pat, zero)


def _finish(h, b1, w2, b2):
    def body(h_ref, b1_ref, w2_ref, b2_ref, o_ref):
        a = jnp.maximum(h_ref[...] + b1_ref[...], 0.0).astype(jnp.bfloat16)
        o = jnp.dot(a, w2_ref[...], preferred_element_type=jnp.float32)
        o_ref[...] = o + b2_ref[...]

    return pl.pallas_call(
        body,
        grid=(B // BLK,),
        in_specs=[
            pl.BlockSpec((BLK, H), lambda i: (i, 0)),
            pl.BlockSpec((1, H), lambda i: (0, 0)),
            pl.BlockSpec((H, D), lambda i: (0, 0)),
            pl.BlockSpec((1, D), lambda i: (0, 0)),
        ],
        out_specs=pl.BlockSpec((BLK, D), lambda i: (i, 0)),
        out_shape=jax.ShapeDtypeStruct((B, D), jnp.float32),
    )(h, b1.reshape(1, H), w2, b2.reshape(1, D))


def kernel(idx, tables, W1, b1, W2, b2):
    idx = idx.astype(jnp.int32)
    flat_idx = (idx + (jnp.arange(F, dtype=jnp.int32) * V)[None, :]).reshape(B * F)
    w1 = W1.astype(jnp.bfloat16).reshape(F, D, H)
    proj = _project_tables(tables, w1)
    rows_w = B // NW
    nwin = rows_w // ROWS_PER_WIN
    pat = (
        (jnp.arange(NS, dtype=jnp.int32) * rows_w)[:, None, None]
        + (jnp.arange(nwin, dtype=jnp.int32) * ROWS_PER_WIN)[None, :, None]
        + (jnp.arange(GW, dtype=jnp.int32) // F)[None, None, :]
    )
    zero = jnp.zeros((rows_w, H), jnp.float32)
    h = _gather_sum(proj, flat_idx, pat, zero)
    return _finish(h, b1, W2.astype(jnp.bfloat16), b2)
